# Initial kernel scaffold; baseline (speedup 1.0000x reference)
import jax
import jax.numpy as jnp
from jax.experimental import pallas as pl


def kernel(x, weight, bias):
    raise NotImplementedError("write your pallas kernel here")


# speedup vs baseline: 20.9599x; 20.9599x over previous
import jax
import jax.numpy as jnp
from jax import lax
from jax.experimental import pallas as pl
from jax.experimental.pallas import tpu as pltpu

_EPS = 1e-5
_W = 128


def _body(x_ref, w_ref, b_ref, o_ref):
    T, D = x_ref.shape[1], x_ref.shape[2]
    C = _W
    NC = T // C

    rows = lax.broadcasted_iota(jnp.int32, (C, C), 0)
    cols = lax.broadcasted_iota(jnp.int32, (C, C), 1)
    tri = jnp.where(cols <= rows, 1.0, 0.0).astype(jnp.float32)

    u1 = lax.broadcasted_iota(jnp.int32, (C, 1), 0).astype(jnp.float32) + 1.0
    inv_first = 1.0 / u1
    inv_full = jnp.float32(1.0 / _W)

    wvec = w_ref[...]
    bvec = b_ref[...]

    prev_q = None
    for k in range(NC):
        ck = x_ref[0, k * C:(k + 1) * C, :]
        cat = jnp.concatenate([ck, ck * ck], axis=1)
        p = jnp.dot(tri, cat, preferred_element_type=jnp.float32)
        tot = p[C - 1:C, :]
        q = tot - p
        if k == 0:
            win = p
            inv = inv_first
        else:
            win = p + prev_q
            inv = inv_full
        prev_q = q

        s1 = win[:, :D]
        s2 = win[:, D:]
        loc = s1 * inv
        var = s2 * inv - loc * loc
        y = (ck - loc) * lax.rsqrt(var + _EPS)
        o_ref[0, k * C:(k + 1) * C, :] = y * wvec + bvec


def kernel(x, weight, bias):
    B, T, D = x.shape
    w2 = weight.reshape(1, D).astype(jnp.float32)
    b2 = bias.reshape(1, D).astype(jnp.float32)
    return pl.pallas_call(
        _body,
        out_shape=jax.ShapeDtypeStruct((B, T, D), jnp.float32),
        grid=(B,),
        in_specs=[
            pl.BlockSpec((1, T, D), lambda b: (b, 0, 0)),
            pl.BlockSpec((1, D), lambda b: (0, 0)),
            pl.BlockSpec((1, D), lambda b: (0, 0)),
        ],
        out_specs=pl.BlockSpec((1, T, D), lambda b: (b, 0, 0)),
        compiler_params=pltpu.CompilerParams(
            dimension_semantics=("parallel",),
            vmem_limit_bytes=48 * 1024 * 1024,
        ),
        name="temporal_norm",
    )(x, w2, b2)


# speedup vs baseline: 22.6603x; 1.0811x over previous
import jax
import jax.numpy as jnp
from jax import lax
from jax.experimental import pallas as pl
from jax.experimental.pallas import tpu as pltpu

_EPS = 1e-5
_W = 128


def _body(x_ref, w_ref, b_ref, o_ref):
    T, D = x_ref.shape[1], x_ref.shape[2]
    C = _W
    NC = T // C

    rows = lax.broadcasted_iota(jnp.int32, (C, 2 * C), 0)
    cols = lax.broadcasted_iota(jnp.int32, (C, 2 * C), 1)
    band = jnp.where((cols > rows) & (cols <= rows + C), 1.0, 0.0)
    band = band.astype(jnp.float32)

    u1 = lax.broadcasted_iota(jnp.int32, (C, 1), 0).astype(jnp.float32) + 1.0
    inv_first = 1.0 / u1
    inv_full = jnp.float32(1.0 / _W)

    wvec = w_ref[...]
    bvec = b_ref[...]

    prev_cat = jnp.zeros((C, 2 * D), jnp.float32)
    for k in range(NC):
        ck = x_ref[0, k * C:(k + 1) * C, :]
        cat = jnp.concatenate([ck, ck * ck], axis=1)
        xe = jnp.concatenate([prev_cat, cat], axis=0)
        win = jnp.dot(band, xe, preferred_element_type=jnp.float32)
        prev_cat = cat
        inv = inv_first if k == 0 else inv_full

        s1 = win[:, :D]
        s2 = win[:, D:]
        loc = s1 * inv
        var = s2 * inv - loc * loc
        y = (ck - loc) * lax.rsqrt(var + _EPS)
        o_ref[0, k * C:(k + 1) * C, :] = y * wvec + bvec


def kernel(x, weight, bias):
    B, T, D = x.shape
    w2 = weight.reshape(1, D).astype(jnp.float32)
    b2 = bias.reshape(1, D).astype(jnp.float32)
    return pl.pallas_call(
        _body,
        out_shape=jax.ShapeDtypeStruct((B, T, D), jnp.float32),
        grid=(B,),
        in_specs=[
            pl.BlockSpec((1, T, D), lambda b: (b, 0, 0)),
            pl.BlockSpec((1, D), lambda b: (0, 0)),
            pl.BlockSpec((1, D), lambda b: (0, 0)),
        ],
        out_specs=pl.BlockSpec((1, T, D), lambda b: (b, 0, 0)),
        compiler_params=pltpu.CompilerParams(
            dimension_semantics=("parallel",),
            vmem_limit_bytes=48 * 1024 * 1024,
        ),
        name="temporal_norm",
    )(x, w2, b2)


# speedup vs baseline: 24.5332x; 1.0827x over previous
import jax
import jax.numpy as jnp
from jax import lax
from jax.experimental import pallas as pl
from jax.experimental.pallas import tpu as pltpu

_EPS = 1e-5
_W = 128


def _body(x_ref, w_ref, b_ref, o_ref):
    T, D = x_ref.shape[1], x_ref.shape[2]
    C = _W
    NC = T // C

    rows = lax.broadcasted_iota(jnp.int32, (C, 2 * C), 0)
    cols = lax.broadcasted_iota(jnp.int32, (C, 2 * C), 1)
    band = jnp.where((cols > rows) & (cols <= rows + C), 1.0, 0.0)
    band = band.astype(jnp.float32)

    u1 = lax.broadcasted_iota(jnp.int32, (C, 1), 0).astype(jnp.float32) + 1.0
    inv_first = 1.0 / u1
    band_full = band * jnp.float32(1.0 / _W)

    wvec = w_ref[...]
    bvec = b_ref[...]

    prev_cat = jnp.zeros((C, 2 * D), jnp.float32)
    for k in range(NC):
        ck = x_ref[0, k * C:(k + 1) * C, :]
        cat = jnp.concatenate([ck, ck * ck], axis=1)
        xe = jnp.concatenate([prev_cat, cat], axis=0)
        if k == 0:
            win = jnp.dot(band, xe, preferred_element_type=jnp.float32)
            win = win * inv_first
        else:
            win = jnp.dot(band_full, xe, preferred_element_type=jnp.float32)
        prev_cat = cat

        loc = win[:, :D]
        m2 = win[:, D:]
        var = m2 - loc * loc
        y = (ck - loc) * lax.rsqrt(var + _EPS)
        o_ref[0, k * C:(k + 1) * C, :] = y * wvec + bvec


def kernel(x, weight, bias):
    B, T, D = x.shape
    w2 = weight.reshape(1, D).astype(jnp.float32)
    b2 = bias.reshape(1, D).astype(jnp.float32)
    return pl.pallas_call(
        _body,
        out_shape=jax.ShapeDtypeStruct((B, T, D), jnp.float32),
        grid=(B,),
        in_specs=[
            pl.BlockSpec((1, T, D), lambda b: (b, 0, 0)),
            pl.BlockSpec((1, D), lambda b: (0, 0)),
            pl.BlockSpec((1, D), lambda b: (0, 0)),
        ],
        out_specs=pl.BlockSpec((1, T, D), lambda b: (b, 0, 0)),
        compiler_params=pltpu.CompilerParams(
            dimension_semantics=("parallel",),
            vmem_limit_bytes=48 * 1024 * 1024,
        ),
        name="temporal_norm",
    )(x, w2, b2)


# speedup vs baseline: 25.1937x; 1.0269x over previous
import jax
import jax.numpy as jnp
from jax import lax
from jax.experimental import pallas as pl
from jax.experimental.pallas import tpu as pltpu

_EPS = 1e-5
_W = 128


def _body(x_ref, w_ref, b_ref, o_ref):
    T, D = x_ref.shape[1], x_ref.shape[2]
    C = _W
    NC = T // C

    rows = lax.broadcasted_iota(jnp.int32, (C, 2 * C), 0)
    cols = lax.broadcasted_iota(jnp.int32, (C, 2 * C), 1)
    band = jnp.where((cols > rows) & (cols <= rows + C), 1.0, 0.0)
    band = band.astype(jnp.float32)

    u1 = lax.broadcasted_iota(jnp.int32, (C, 1), 0).astype(jnp.float32) + 1.0
    inv_first = 1.0 / u1
    tri = band[:, C:]
    band_full_b = (band * jnp.float32(1.0 / _W)).astype(jnp.bfloat16)

    wvec = w_ref[...]
    bvec = b_ref[...]

    prev_cat = None
    for k in range(NC):
        ck = x_ref[0, k * C:(k + 1) * C, :]
        ckb = ck.astype(jnp.bfloat16)
        cat = jnp.concatenate([ckb, ckb * ckb], axis=1)
        if k == 0:
            catf = jnp.concatenate([ck, ck * ck], axis=1)
            win = jnp.dot(tri, catf, preferred_element_type=jnp.float32)
            win = win * inv_first
        else:
            xe = jnp.concatenate([prev_cat, cat], axis=0)
            win = jnp.dot(band_full_b, xe, preferred_element_type=jnp.float32)
        prev_cat = cat

        loc = win[:, :D]
        m2 = win[:, D:]
        var = m2 - loc * loc
        y = (ck - loc) * lax.rsqrt(var + _EPS)
        o_ref[0, k * C:(k + 1) * C, :] = y * wvec + bvec


def kernel(x, weight, bias):
    B, T, D = x.shape
    w2 = weight.reshape(1, D).astype(jnp.float32)
    b2 = bias.reshape(1, D).astype(jnp.float32)
    return pl.pallas_call(
        _body,
        out_shape=jax.ShapeDtypeStruct((B, T, D), jnp.float32),
        grid=(B,),
        in_specs=[
            pl.BlockSpec((1, T, D), lambda b: (b, 0, 0)),
            pl.BlockSpec((1, D), lambda b: (0, 0)),
            pl.BlockSpec((1, D), lambda b: (0, 0)),
        ],
        out_specs=pl.BlockSpec((1, T, D), lambda b: (b, 0, 0)),
        compiler_params=pltpu.CompilerParams(
            dimension_semantics=("parallel",),
            vmem_limit_bytes=48 * 1024 * 1024,
        ),
        name="temporal_norm",
    )(x, w2, b2)


# speedup vs baseline: 26.4254x; 1.0489x over previous
import jax
import jax.numpy as jnp
from jax import lax
from jax.experimental import pallas as pl
from jax.experimental.pallas import tpu as pltpu

_EPS = 1e-5
_W = 128


def _body(x_ref, w_ref, b_ref, o_ref):
    NB, T, D = x_ref.shape
    C = _W
    NC = T // C

    rows = lax.broadcasted_iota(jnp.int32, (C, 2 * C), 0)
    cols = lax.broadcasted_iota(jnp.int32, (C, 2 * C), 1)
    band = jnp.where((cols > rows) & (cols <= rows + C), 1.0, 0.0)
    band = band.astype(jnp.float32)

    u1 = lax.broadcasted_iota(jnp.int32, (C, 1), 0).astype(jnp.float32) + 1.0
    inv_first = 1.0 / u1
    tri = band[:, C:]
    band_full_b = (band * jnp.float32(1.0 / _W)).astype(jnp.bfloat16)

    wvec = w_ref[...]
    bvec = b_ref[...]

    for n in range(NB):
        prev_cat = None
        for k in range(NC):
            ck = x_ref[n, k * C:(k + 1) * C, :]
            ckb = ck.astype(jnp.bfloat16)
            cat = jnp.concatenate([ckb, ckb * ckb], axis=1)
            if k == 0:
                catf = jnp.concatenate([ck, ck * ck], axis=1)
                win = jnp.dot(tri, catf, preferred_element_type=jnp.float32)
                win = win * inv_first
            else:
                xe = jnp.concatenate([prev_cat, cat], axis=0)
                win = jnp.dot(band_full_b, xe,
                              preferred_element_type=jnp.float32)
            prev_cat = cat

            loc = win[:, :D]
            m2 = win[:, D:]
            var = m2 - loc * loc
            y = (ck - loc) * lax.rsqrt(var + _EPS)
            o_ref[n, k * C:(k + 1) * C, :] = y * wvec + bvec


def kernel(x, weight, bias):
    B, T, D = x.shape
    NB = 2
    w2 = weight.reshape(1, D).astype(jnp.float32)
    b2 = bias.reshape(1, D).astype(jnp.float32)
    return pl.pallas_call(
        _body,
        out_shape=jax.ShapeDtypeStruct((B, T, D), jnp.float32),
        grid=(B // NB,),
        in_specs=[
            pl.BlockSpec((NB, T, D), lambda b: (b, 0, 0)),
            pl.BlockSpec((1, D), lambda b: (0, 0)),
            pl.BlockSpec((1, D), lambda b: (0, 0)),
        ],
        out_specs=pl.BlockSpec((NB, T, D), lambda b: (b, 0, 0)),
        compiler_params=pltpu.CompilerParams(
            dimension_semantics=("parallel",),
            vmem_limit_bytes=48 * 1024 * 1024,
        ),
        name="temporal_norm",
    )(x, w2, b2)
